# Initial kernel scaffold; baseline (speedup 1.0000x reference)
import jax
import jax.numpy as jnp
from jax.experimental import pallas as pl


def kernel(x, edge_index, edge_type, edge_weight, seed_node_id, W_ft, b_ft, W00, b00, g00, be00, W01, b01, g01, be01, W10, b10, g10, be10, W11, b11, g11, be11, Wp1, bp1, Wp2, bp2, Wp3, bp3):
    raise NotImplementedError("write your pallas kernel here")


# speedup vs baseline: 1.0938x; 1.0938x over previous
import jax
import jax.numpy as jnp
from jax.experimental import pallas as pl
from jax.experimental.pallas import tpu as pltpu

N = 10000
H = 64
EPS = 1e-5
ROWS = 1000


def _mm_relu_kernel(x_ref, w_ref, b_ref, o_ref):
    o_ref[...] = jax.nn.relu(
        jnp.dot(x_ref[...], w_ref[...], preferred_element_type=jnp.float32)
        + b_ref[...]
    )


def _mm_relu(x, w, b):
    m, k = x.shape
    h = w.shape[1]
    return pl.pallas_call(
        _mm_relu_kernel,
        grid=(m // ROWS,),
        in_specs=[
            pl.BlockSpec((ROWS, k), lambda i: (i, 0)),
            pl.BlockSpec((k, h), lambda i: (0, 0)),
            pl.BlockSpec((h,), lambda i: (0,)),
        ],
        out_specs=pl.BlockSpec((ROWS, h), lambda i: (i, 0)),
        out_shape=jax.ShapeDtypeStruct((m, h), jnp.float32),
    )(x, w, b)


def _bn(x, g, b):
    return g * (x / jnp.sqrt(1.0 + EPS)) + b


def _gcn(x, src, dst, ew, W, b, n):
    loop = jnp.arange(n, dtype=src.dtype)
    s = jnp.concatenate([src, loop])
    d = jnp.concatenate([dst, loop])
    w = jnp.concatenate([ew, jnp.ones((n,), x.dtype)])
    deg = jnp.zeros((n,), x.dtype).at[d].add(w)
    dis = jnp.where(deg > 0, 1.0 / jnp.sqrt(deg), 0.0)
    norm = dis[s] * w * dis[d]
    xw = x @ W
    out = jnp.zeros((n, xw.shape[1]), x.dtype).at[d].add(xw[s] * norm[:, None])
    return out + b


def _block(h, src, dst, ew, Wa, ba, ga, bea, Wb, bb, gb, beb, n):
    o = jax.nn.relu(_bn(_gcn(h, src, dst, ew, Wa, ba, n), ga, bea))
    return _bn(_gcn(o, src, dst, ew, Wb, bb, n), gb, beb) + o


def kernel(x, edge_index, edge_type, edge_weight, seed_node_id, W_ft, b_ft, W00, b00, g00, be00, W01, b01, g01, be01, W10, b10, g10, be10, W11, b11, g11, be11, Wp1, bp1, Wp2, bp2, Wp3, bp3):
    n = x.shape[0]
    h = _mm_relu(x, W_ft, b_ft)
    m0 = (edge_type == 0).astype(edge_weight.dtype)
    m1 = (edge_type == 1).astype(edge_weight.dtype)
    src = edge_index[0]
    dst = edge_index[1]
    x0 = _block(h, src, dst, edge_weight * m0, W00, b00, g00, be00, W01, b01, g01, be01, n)
    x1 = _block(h, src, dst, edge_weight * m1, W10, b10, g10, be10, W11, b11, g11, be11, n)
    s0 = jnp.broadcast_to(x0[seed_node_id], (n, x0.shape[1]))
    s1 = jnp.broadcast_to(x1[seed_node_id], (n, x1.shape[1]))
    xc = jnp.concatenate([x0, x1, s0, s1], axis=1)
    o = jax.nn.relu(xc @ Wp1 + bp1)
    o = jax.nn.relu(o @ Wp2 + bp2)
    o = o @ Wp3 + bp3
    return o.squeeze(-1)


# speedup vs baseline: 13.7864x; 12.6040x over previous
import functools

import jax
import jax.numpy as jnp
from jax import lax
from jax.experimental import pallas as pl
from jax.experimental.pallas import tpu as pltpu
from jax.experimental.pallas import tpu_sc as plsc

N = 10000
NPAD = 10240
E = 320000
H = 64
H2 = 2 * H
EPS = 1e-5
NT = 16
NCORE = 2
CH = 128
ECH = 80
EPT = ECH * CH
E_PAD = EPT * NT * NCORE
NPT = NPAD // NT
ROWS = 1024

_mesh = plsc.VectorSubcoreMesh(core_axis_name="c", subcore_axis_name="s")


@functools.partial(
    pl.kernel,
    out_type=[
        jax.ShapeDtypeStruct((NCORE * 2 * NPAD,), jnp.float32),
        jax.ShapeDtypeStruct((NCORE * NT * ECH, CH), jnp.float32),
        jax.ShapeDtypeStruct((NCORE * NT * ECH, CH), jnp.float32),
    ],
    mesh=_mesh,
    scratch_types=[
        pltpu.VMEM((ECH, CH), jnp.int32),
        pltpu.VMEM((ECH, CH), jnp.int32),
        pltpu.VMEM((ECH, CH), jnp.float32),
        pltpu.VMEM((ECH, CH), jnp.int32),
        pltpu.VMEM((ECH, CH), jnp.float32),
        pltpu.VMEM((ECH, CH), jnp.float32),
        pltpu.VMEM((2 * NPAD // NT,), jnp.float32),
        pltpu.VMEM_SHARED((2 * NPAD,), jnp.float32),
    ],
)
def _sc_deg(dst_hbm, ew_hbm, et_hbm, deg_hbm, ewm0_hbm, ewm1_hbm,
            dstv, dstv1, ewv, etv, ewm0v, ewm1v, zv, sh_deg):
    c = lax.axis_index("c")
    s = lax.axis_index("s")
    w = c * NT + s
    npt2 = 2 * NPAD // NT
    pltpu.sync_copy(dst_hbm.at[pl.ds(w * ECH, ECH)], dstv)
    pltpu.sync_copy(ew_hbm.at[pl.ds(w * ECH, ECH)], ewv)
    pltpu.sync_copy(et_hbm.at[pl.ds(w * ECH, ECH)], etv)
    for k in range(npt2 // 16):
        zv[pl.ds(k * 16, 16)] = jnp.zeros((16,), jnp.float32)
    pltpu.sync_copy(zv, sh_deg.at[pl.ds(s * npt2, npt2)])

    def mask_body(j, carry):
        for k in range(CH // 16):
            sl = pl.ds(k * 16, 16)
            et16 = etv[j, sl]
            ew16 = ewv[j, sl]
            zero = jnp.zeros((16,), jnp.float32)
            ewm0v[j, sl] = jnp.where(et16 == 0, ew16, zero)
            ewm1v[j, sl] = jnp.where(et16 == 1, ew16, zero)
            dstv1[j, sl] = dstv[j, sl] + NPAD
        return carry
    lax.fori_loop(0, ECH, mask_body, 0)
    pltpu.sync_copy(ewm0v, ewm0_hbm.at[pl.ds(w * ECH, ECH)])
    pltpu.sync_copy(ewm1v, ewm1_hbm.at[pl.ds(w * ECH, ECH)])
    plsc.subcore_barrier()

    def add_body(j, carry):
        pltpu.sync_copy(ewm0v.at[j], sh_deg.at[dstv.at[j]], add=True)
        pltpu.sync_copy(ewm1v.at[j], sh_deg.at[dstv1.at[j]], add=True)
        return carry
    lax.fori_loop(0, ECH, add_body, 0)
    plsc.subcore_barrier()
    pltpu.sync_copy(sh_deg.at[pl.ds(s * npt2, npt2)],
                    deg_hbm.at[pl.ds(c * 2 * NPAD + s * npt2, npt2)])


@functools.partial(
    pl.kernel,
    out_type=jax.ShapeDtypeStruct((NCORE * NPAD, H2), jnp.float32),
    mesh=_mesh,
    scratch_types=[
        pltpu.VMEM((8, CH), jnp.int32),
        pltpu.VMEM((8, CH), jnp.int32),
        pltpu.VMEM((8, CH), jnp.float32),
        pltpu.VMEM((8, CH), jnp.float32),
        pltpu.VMEM((CH, H2), jnp.float32),
        pltpu.VMEM_SHARED((NPAD, H2), jnp.float32),
        pltpu.SemaphoreType.DMA,
    ],
)
def _sc_agg(src_hbm, dst_hbm, ewm0_hbm, ewm1_hbm, y_hbm, z_hbm,
            src8, dst8, e08, e18, rows, sh_z, sem):
    c = lax.axis_index("c")
    s = lax.axis_index("s")
    w = c * NT + s

    def zero_body(r, carry):
        for kk in range(H2 // 16):
            rows[r, pl.ds(kk * 16, 16)] = jnp.zeros((16,), jnp.float32)
        return carry
    lax.fori_loop(0, CH, zero_body, 0)
    for k in range(NPT // CH):
        pltpu.sync_copy(rows, sh_z.at[pl.ds(s * NPT + k * CH, CH)])
    plsc.subcore_barrier()

    def group_body(g, carry):
        base = w * ECH + g * 8
        pltpu.sync_copy(src_hbm.at[pl.ds(base, 8)], src8)
        pltpu.sync_copy(dst_hbm.at[pl.ds(base, 8)], dst8)
        pltpu.sync_copy(ewm0_hbm.at[pl.ds(base, 8)], e08)
        pltpu.sync_copy(ewm1_hbm.at[pl.ds(base, 8)], e18)

        def chunk_body(j, carry2):
            pltpu.async_copy(y_hbm.at[src8.at[j]], rows, sem).wait()
            for gg in range(CH // 16):
                e0 = e08[j, pl.ds(gg * 16, 16)]
                e1 = e18[j, pl.ds(gg * 16, 16)]
                for l in range(16):
                    r = gg * 16 + l
                    sp0 = jnp.full((16,), e0[l], jnp.float32)
                    sp1 = jnp.full((16,), e1[l], jnp.float32)
                    for kk in range(H // 16):
                        rows[r, pl.ds(kk * 16, 16)] = (
                            rows[r, pl.ds(kk * 16, 16)] * sp0)
                    for kk in range(H // 16):
                        rows[r, pl.ds(H + kk * 16, 16)] = (
                            rows[r, pl.ds(H + kk * 16, 16)] * sp1)
            pltpu.sync_copy(rows, sh_z.at[dst8.at[j]], add=True)
            return carry2
        lax.fori_loop(0, 8, chunk_body, 0)
        return carry
    lax.fori_loop(0, ECH // 8, group_body, 0)
    plsc.subcore_barrier()
    pltpu.sync_copy(sh_z.at[pl.ds(s * NPT, NPT)],
                    z_hbm.at[pl.ds(c * NPAD + s * NPT, NPT)])


def _tc_call(body, grid, in_specs, out_specs, out_shape):
    return pl.pallas_call(
        body, grid=grid, in_specs=in_specs, out_specs=out_specs,
        out_shape=out_shape)


def _row_spec(d):
    return pl.BlockSpec((ROWS, d), lambda i: (i, 0))


def _row2_spec(d):
    return pl.BlockSpec((NCORE, ROWS, d), lambda i: (0, i, 0))


def _full_spec(shape):
    return pl.BlockSpec(shape, lambda i: (0,) * len(shape))


def _tc_h_body(x_ref, w_ref, b_ref, o_ref):
    o_ref[...] = jax.nn.relu(
        jnp.dot(x_ref[...], w_ref[...], preferred_element_type=jnp.float32)
        + b_ref[...])


def _tc_b_body(h_ref, deg_ref, w0_ref, w1_ref, y_ref, dis_ref):
    deg = deg_ref[0] + deg_ref[1]
    dis = 1.0 / jnp.sqrt(deg + 1.0)
    h = h_ref[...]
    y_ref[:, 0:H] = dis[0][:, None] * jnp.dot(
        h, w0_ref[...], preferred_element_type=jnp.float32)
    y_ref[:, H:H2] = dis[1][:, None] * jnp.dot(
        h, w1_ref[...], preferred_element_type=jnp.float32)
    dis_ref[...] = dis


def _tc_c_body(z_ref, y_ref, dis_ref, b_ref, g_ref, be_ref, w_ref,
               o_ref, y2_ref):
    inv = 1.0 / jnp.sqrt(1.0 + EPS)
    zz = z_ref[0] + z_ref[1] + y_ref[...]
    for t in range(2):
        dis = dis_ref[t][:, None]
        agg = dis * zz[:, t * H:(t + 1) * H] + b_ref[t]
        o = jax.nn.relu(g_ref[t] * (agg * inv) + be_ref[t])
        o_ref[t] = o
        y2_ref[:, t * H:(t + 1) * H] = dis * jnp.dot(
            o, w_ref[t], preferred_element_type=jnp.float32)


def _tc_d_body(z_ref, y2_ref, o_ref, dis_ref, b_ref, g_ref, be_ref, x_ref):
    inv = 1.0 / jnp.sqrt(1.0 + EPS)
    zz = z_ref[0] + z_ref[1] + y2_ref[...]
    for t in range(2):
        dis = dis_ref[t][:, None]
        agg = dis * zz[:, t * H:(t + 1) * H] + b_ref[t]
        x_ref[t] = g_ref[t] * (agg * inv) + be_ref[t] + o_ref[t]


def _tc_s_body(x_ref, seed_ref, wcd_ref, bp1_ref, sb_ref):
    sd = seed_ref[0]
    x0r = x_ref[0, pl.ds(sd, 1), :]
    x1r = x_ref[1, pl.ds(sd, 1), :]
    sb_ref[...] = (
        jnp.dot(x0r, wcd_ref[0], preferred_element_type=jnp.float32)
        + jnp.dot(x1r, wcd_ref[1], preferred_element_type=jnp.float32)
        + bp1_ref[...])


def _tc_e_body(x_ref, sb_ref, wab_ref, w2_ref, b2_ref, w3_ref, b3_ref, o_ref):
    o = jax.nn.relu(
        jnp.dot(x_ref[0], wab_ref[0], preferred_element_type=jnp.float32)
        + jnp.dot(x_ref[1], wab_ref[1], preferred_element_type=jnp.float32)
        + sb_ref[...])
    o = jax.nn.relu(jnp.dot(o, w2_ref[...], preferred_element_type=jnp.float32)
                    + b2_ref[...])
    o_ref[...] = jnp.dot(o, w3_ref[...], preferred_element_type=jnp.float32) \
        + b3_ref[...]


def kernel(x, edge_index, edge_type, edge_weight, seed_node_id, W_ft, b_ft, W00, b00, g00, be00, W01, b01, g01, be01, W10, b10, g10, be10, W11, b11, g11, be11, Wp1, bp1, Wp2, bp2, Wp3, bp3):
    f32 = jnp.float32
    grid = (NPAD // ROWS,)

    xp = jnp.pad(x, ((0, NPAD - N), (0, 0)))
    nrow = NCORE * NT * ECH
    src = jnp.pad(edge_index[0], (0, E_PAD - E)).reshape(nrow, CH)
    dst = jnp.pad(edge_index[1], (0, E_PAD - E)).reshape(nrow, CH)
    ew = jnp.pad(edge_weight, (0, E_PAD - E)).reshape(nrow, CH)
    et = jnp.pad(edge_type, (0, E_PAD - E),
                 constant_values=-1).reshape(nrow, CH)
    seed = jnp.asarray(seed_node_id, jnp.int32).reshape(1)
    bL0 = jnp.stack([b00, b10]); gL0 = jnp.stack([g00, g10])
    beL0 = jnp.stack([be00, be10])
    WL1 = jnp.stack([W01, W11])
    bL1 = jnp.stack([b01, b11]); gL1 = jnp.stack([g01, g11])
    beL1 = jnp.stack([be01, be11])
    Wp1ab = jnp.stack([Wp1[0:H], Wp1[H:2 * H]])
    Wp1cd = jnp.stack([Wp1[2 * H:3 * H], Wp1[3 * H:4 * H]])

    deg, ewm0, ewm1 = _sc_deg(dst, ew, et)
    deg4 = deg.reshape(NCORE, 2, NPAD)

    h = _tc_call(
        _tc_h_body, grid,
        [_row_spec(128), _full_spec((128, H)), _full_spec((H,))],
        _row_spec(H), jax.ShapeDtypeStruct((NPAD, H), f32))(xp, W_ft, b_ft)

    y, dis = _tc_call(
        _tc_b_body, grid,
        [_row_spec(H), pl.BlockSpec((NCORE, 2, ROWS), lambda i: (0, 0, i)),
         _full_spec((H, H)), _full_spec((H, H))],
        [_row_spec(H2), pl.BlockSpec((2, ROWS), lambda i: (0, i))],
        [jax.ShapeDtypeStruct((NPAD, H2), f32),
         jax.ShapeDtypeStruct((2, NPAD), f32)])(h, deg4, W00, W10)

    z = _sc_agg(src, dst, ewm0, ewm1, y)
    z = z.reshape(NCORE, NPAD, H2)

    o, y2 = _tc_call(
        _tc_c_body, grid,
        [_row2_spec(H2), _row_spec(H2),
         pl.BlockSpec((2, ROWS), lambda i: (0, i)),
         _full_spec((2, H)), _full_spec((2, H)),
         _full_spec((2, H)), _full_spec((2, H, H))],
        [_row2_spec(H), _row_spec(H2)],
        [jax.ShapeDtypeStruct((NCORE, NPAD, H), f32),
         jax.ShapeDtypeStruct((NPAD, H2), f32)])(
             z, y, dis, bL0, gL0, beL0, WL1)

    z2 = _sc_agg(src, dst, ewm0, ewm1, y2)
    z2 = z2.reshape(NCORE, NPAD, H2)

    x01 = _tc_call(
        _tc_d_body, grid,
        [_row2_spec(H2), _row_spec(H2), _row2_spec(H),
         pl.BlockSpec((2, ROWS), lambda i: (0, i)),
         _full_spec((2, H)), _full_spec((2, H)), _full_spec((2, H))],
        _row2_spec(H),
        jax.ShapeDtypeStruct((NCORE, NPAD, H), f32))(
            z2, y2, o, dis, bL1, gL1, beL1)

    sb = pl.pallas_call(
        _tc_s_body,
        grid=(1,),
        in_specs=[_full_spec((NCORE, NPAD, H)),
                  pl.BlockSpec(memory_space=pltpu.SMEM),
                  _full_spec((NCORE, H, H)), _full_spec((H,))],
        out_specs=_full_spec((1, H)),
        out_shape=jax.ShapeDtypeStruct((1, H), f32))(x01, seed, Wp1cd, bp1)

    res = _tc_call(
        _tc_e_body, grid,
        [_row2_spec(H), _full_spec((1, H)), _full_spec((NCORE, H, H)),
         _full_spec((H, H)), _full_spec((H,)), _full_spec((H, 1)),
         _full_spec((1,))],
        _row_spec(1),
        jax.ShapeDtypeStruct((NPAD, 1), f32))(
            x01, sb, Wp1ab, Wp2, bp2, Wp3, bp3)

    return res[:N, 0]


# speedup vs baseline: 14.5262x; 1.0537x over previous
import functools

import jax
import jax.numpy as jnp
from jax import lax
from jax.experimental import pallas as pl
from jax.experimental.pallas import tpu as pltpu
from jax.experimental.pallas import tpu_sc as plsc

N = 10000
NPAD = 10240
E = 320000
H = 64
H2 = 2 * H
EPS = 1e-5
NT = 16
NCORE = 2
CH = 64
ECH = 160
GSL = 16
NG = ECH // GSL
EPT = ECH * CH
E_PAD = EPT * NT * NCORE
NPT = NPAD // NT
ROWS = 1024

_mesh = plsc.VectorSubcoreMesh(core_axis_name="c", subcore_axis_name="s")


@functools.partial(
    pl.kernel,
    out_type=[
        jax.ShapeDtypeStruct((NCORE * 2 * NPAD,), jnp.float32),
        jax.ShapeDtypeStruct((E_PAD // CH, CH), jnp.float32),
        jax.ShapeDtypeStruct((E_PAD // CH, CH), jnp.float32),
    ],
    mesh=_mesh,
    scratch_types=[
        pltpu.VMEM((ECH, CH), jnp.int32),
        pltpu.VMEM((ECH, CH), jnp.int32),
        pltpu.VMEM((ECH, CH), jnp.float32),
        pltpu.VMEM((ECH, CH), jnp.int32),
        pltpu.VMEM((ECH, CH), jnp.float32),
        pltpu.VMEM((ECH, CH), jnp.float32),
        pltpu.VMEM((2 * NPAD // NT,), jnp.float32),
        pltpu.VMEM_SHARED((2 * NPAD,), jnp.float32),
    ],
)
def _sc_deg(dst_hbm, ew_hbm, et_hbm, deg_hbm, ewm0_hbm, ewm1_hbm,
            dstv, dstv1, ewv, etv, ewm0v, ewm1v, zv, sh_deg):
    c = lax.axis_index("c")
    s = lax.axis_index("s")
    w = c * NT + s
    npt2 = 2 * NPAD // NT
    pltpu.sync_copy(dst_hbm.at[pl.ds(w * ECH, ECH)], dstv)
    pltpu.sync_copy(ew_hbm.at[pl.ds(w * ECH, ECH)], ewv)
    pltpu.sync_copy(et_hbm.at[pl.ds(w * ECH, ECH)], etv)
    for k in range(npt2 // 16):
        zv[pl.ds(k * 16, 16)] = jnp.zeros((16,), jnp.float32)
    pltpu.sync_copy(zv, sh_deg.at[pl.ds(s * npt2, npt2)])

    def mask_body(j, carry):
        for k in range(CH // 16):
            sl = pl.ds(k * 16, 16)
            et16 = etv[j, sl]
            ew16 = ewv[j, sl]
            zero = jnp.zeros((16,), jnp.float32)
            ewm0v[j, sl] = jnp.where(et16 == 0, ew16, zero)
            ewm1v[j, sl] = jnp.where(et16 == 1, ew16, zero)
            dstv1[j, sl] = dstv[j, sl] + NPAD
        return carry
    lax.fori_loop(0, ECH, mask_body, 0)
    pltpu.sync_copy(ewm0v, ewm0_hbm.at[pl.ds(w * ECH, ECH)])
    pltpu.sync_copy(ewm1v, ewm1_hbm.at[pl.ds(w * ECH, ECH)])
    plsc.subcore_barrier()

    def add_body(j, carry):
        pltpu.sync_copy(ewm0v.at[j], sh_deg.at[dstv.at[j]], add=True)
        pltpu.sync_copy(ewm1v.at[j], sh_deg.at[dstv1.at[j]], add=True)
        return carry
    lax.fori_loop(0, ECH, add_body, 0)
    plsc.subcore_barrier()
    pltpu.sync_copy(sh_deg.at[pl.ds(s * npt2, npt2)],
                    deg_hbm.at[pl.ds(c * 2 * NPAD + s * npt2, npt2)])


@functools.partial(
    pl.kernel,
    out_type=jax.ShapeDtypeStruct((NCORE * NPAD, H2), jnp.float32),
    mesh=_mesh,
    scratch_types=[
        pltpu.VMEM((2, GSL, CH), jnp.int32),
        pltpu.VMEM((2, GSL, CH), jnp.int32),
        pltpu.VMEM((2, GSL, CH), jnp.float32),
        pltpu.VMEM((2, GSL, CH), jnp.float32),
        pltpu.VMEM((4, CH, H2), jnp.float32),
        pltpu.VMEM_SHARED((NPAD, H2), jnp.float32),
        pltpu.SemaphoreType.DMA((4,)),
        pltpu.SemaphoreType.DMA((4,)),
        pltpu.SemaphoreType.DMA((2,)),
    ],
)
def _sc_agg(src_hbm, dst_hbm, ewm0_hbm, ewm1_hbm, y_hbm, z_hbm,
            src_st, dst_st, e0_st, e1_st, rows, sh_z, g_sem, s_sem, st_sem):
    c = lax.axis_index("c")
    s = lax.axis_index("s")
    w = c * NT + s
    tb = w * ECH

    def zrow(r, carry):
        for kk in range(H2 // 16):
            rows[0, r, pl.ds(kk * 16, 16)] = jnp.zeros((16,), jnp.float32)
        return carry
    lax.fori_loop(0, CH, zrow, 0)
    for k in range(NPT // CH):
        pltpu.sync_copy(rows.at[0], sh_z.at[pl.ds(s * NPT + k * CH, CH)])
    plsc.subcore_barrier()

    def issue_stage(g, p):
        b0 = pl.multiple_of(tb + g * GSL, 8)
        pltpu.async_copy(src_hbm.at[pl.ds(b0, GSL)], src_st.at[p],
                         st_sem.at[p])
        pltpu.async_copy(dst_hbm.at[pl.ds(b0, GSL)], dst_st.at[p],
                         st_sem.at[p])
        pltpu.async_copy(ewm0_hbm.at[pl.ds(b0, GSL)], e0_st.at[p],
                         st_sem.at[p])
        pltpu.async_copy(ewm1_hbm.at[pl.ds(b0, GSL)], e1_st.at[p],
                         st_sem.at[p])

    def drain_stage(p):
        pltpu.make_async_copy(src_hbm.at[pl.ds(0, GSL)], src_st.at[p],
                              st_sem.at[p]).wait()
        pltpu.make_async_copy(dst_hbm.at[pl.ds(0, GSL)], dst_st.at[p],
                              st_sem.at[p]).wait()
        pltpu.make_async_copy(ewm0_hbm.at[pl.ds(0, GSL)], e0_st.at[p],
                              st_sem.at[p]).wait()
        pltpu.make_async_copy(ewm1_hbm.at[pl.ds(0, GSL)], e1_st.at[p],
                              st_sem.at[p]).wait()

    def drain_scatter(bb):
        pltpu.make_async_copy(rows.at[bb], sh_z.at[pl.ds(0, CH)],
                              s_sem.at[bb]).wait()

    issue_stage(0, 0)
    drain_stage(0)
    issue_stage(1, 1)
    pltpu.async_copy(y_hbm.at[src_st.at[0, 0]], rows.at[0], g_sem.at[0])
    pltpu.async_copy(y_hbm.at[src_st.at[0, 1]], rows.at[1], g_sem.at[1])

    def slot(j, carry):
        jm = lax.rem(j, GSL)
        g = lax.div(j, GSL)
        p = lax.rem(g, 2)
        b = lax.rem(j, 4)
        at14 = jm == GSL - 2

        @pl.when(jnp.logical_and(jm == 2,
                                 jnp.logical_and(j >= GSL,
                                                 j < (NG - 1) * GSL)))
        def _():
            issue_stage(g + 1, lax.rem(g + 1, 2))

        @pl.when(jnp.logical_and(at14, j < (NG - 1) * GSL))
        def _():
            drain_stage(lax.rem(g + 1, 2))

        jj = j + 2
        bb = lax.rem(jj, 4)
        pj = lax.rem(lax.div(jj, GSL), 2)
        jjm = lax.rem(jj, GSL)

        @pl.when(jj >= 4)
        def _():
            drain_scatter(bb)

        @pl.when(jj < ECH)
        def _():
            pltpu.async_copy(y_hbm.at[src_st.at[pj, jjm]], rows.at[bb],
                             g_sem.at[bb])

        pltpu.make_async_copy(y_hbm.at[src_st.at[0, 0]], rows.at[b],
                              g_sem.at[b]).wait()

        def srow(gg, carry2):
            e016 = e0_st[p, jm, pl.ds(gg * 16, 16)]
            e116 = e1_st[p, jm, pl.ds(gg * 16, 16)]
            for l in range(16):
                r = gg * 16 + l
                sp0 = jnp.full((16,), e016[l], jnp.float32)
                sp1 = jnp.full((16,), e116[l], jnp.float32)
                for kk in range(H // 16):
                    rows[b, r, pl.ds(kk * 16, 16)] = (
                        rows[b, r, pl.ds(kk * 16, 16)] * sp0)
                for kk in range(H // 16):
                    rows[b, r, pl.ds(H + kk * 16, 16)] = (
                        rows[b, r, pl.ds(H + kk * 16, 16)] * sp1)
            return carry2
        lax.fori_loop(0, CH // 16, srow, 0)

        pltpu.async_copy(rows.at[b], sh_z.at[dst_st.at[p, jm]],
                         s_sem.at[b], add=True)
        return carry
    lax.fori_loop(0, ECH, slot, 0)

    drain_scatter(2)
    drain_scatter(3)
    plsc.subcore_barrier()
    pltpu.sync_copy(sh_z.at[pl.ds(s * NPT, NPT)],
                    z_hbm.at[pl.ds(c * NPAD + s * NPT, NPT)])


def _tc_call(body, grid, in_specs, out_specs, out_shape):
    return pl.pallas_call(
        body, grid=grid, in_specs=in_specs, out_specs=out_specs,
        out_shape=out_shape)


def _row_spec(d):
    return pl.BlockSpec((ROWS, d), lambda i: (i, 0))


def _row2_spec(d):
    return pl.BlockSpec((NCORE, ROWS, d), lambda i: (0, i, 0))


def _full_spec(shape):
    return pl.BlockSpec(shape, lambda i: (0,) * len(shape))


def _tc_h_body(x_ref, w_ref, b_ref, o_ref):
    o_ref[...] = jax.nn.relu(
        jnp.dot(x_ref[...], w_ref[...], preferred_element_type=jnp.float32)
        + b_ref[...])


def _tc_b_body(h_ref, deg_ref, w0_ref, w1_ref, y_ref, dis_ref):
    deg = deg_ref[0] + deg_ref[1]
    dis = 1.0 / jnp.sqrt(deg + 1.0)
    h = h_ref[...]
    y_ref[:, 0:H] = dis[0][:, None] * jnp.dot(
        h, w0_ref[...], preferred_element_type=jnp.float32)
    y_ref[:, H:H2] = dis[1][:, None] * jnp.dot(
        h, w1_ref[...], preferred_element_type=jnp.float32)
    dis_ref[...] = dis


def _tc_c_body(z_ref, y_ref, dis_ref, b_ref, g_ref, be_ref, w_ref,
               o_ref, y2_ref):
    inv = 1.0 / jnp.sqrt(1.0 + EPS)
    zz = z_ref[0] + z_ref[1] + y_ref[...]
    for t in range(2):
        dis = dis_ref[t][:, None]
        agg = dis * zz[:, t * H:(t + 1) * H] + b_ref[t]
        o = jax.nn.relu(g_ref[t] * (agg * inv) + be_ref[t])
        o_ref[t] = o
        y2_ref[:, t * H:(t + 1) * H] = dis * jnp.dot(
            o, w_ref[t], preferred_element_type=jnp.float32)


def _tc_d_body(z_ref, y2_ref, o_ref, dis_ref, b_ref, g_ref, be_ref, x_ref):
    inv = 1.0 / jnp.sqrt(1.0 + EPS)
    zz = z_ref[0] + z_ref[1] + y2_ref[...]
    for t in range(2):
        dis = dis_ref[t][:, None]
        agg = dis * zz[:, t * H:(t + 1) * H] + b_ref[t]
        x_ref[t] = g_ref[t] * (agg * inv) + be_ref[t] + o_ref[t]


def _tc_s_body(x_ref, seed_ref, wcd_ref, bp1_ref, sb_ref):
    sd = seed_ref[0]
    x0r = x_ref[0, pl.ds(sd, 1), :]
    x1r = x_ref[1, pl.ds(sd, 1), :]
    sb_ref[...] = (
        jnp.dot(x0r, wcd_ref[0], preferred_element_type=jnp.float32)
        + jnp.dot(x1r, wcd_ref[1], preferred_element_type=jnp.float32)
        + bp1_ref[...])


def _tc_e_body(x_ref, sb_ref, wab_ref, w2_ref, b2_ref, w3_ref, b3_ref, o_ref):
    o = jax.nn.relu(
        jnp.dot(x_ref[0], wab_ref[0], preferred_element_type=jnp.float32)
        + jnp.dot(x_ref[1], wab_ref[1], preferred_element_type=jnp.float32)
        + sb_ref[...])
    o = jax.nn.relu(jnp.dot(o, w2_ref[...], preferred_element_type=jnp.float32)
                    + b2_ref[...])
    o_ref[...] = jnp.dot(o, w3_ref[...], preferred_element_type=jnp.float32) \
        + b3_ref[...]


def kernel(x, edge_index, edge_type, edge_weight, seed_node_id, W_ft, b_ft, W00, b00, g00, be00, W01, b01, g01, be01, W10, b10, g10, be10, W11, b11, g11, be11, Wp1, bp1, Wp2, bp2, Wp3, bp3):
    f32 = jnp.float32
    grid = (NPAD // ROWS,)

    xp = jnp.pad(x, ((0, NPAD - N), (0, 0)))
    nrow = E_PAD // CH
    src = jnp.pad(edge_index[0], (0, E_PAD - E)).reshape(nrow, CH)
    dst = jnp.pad(edge_index[1], (0, E_PAD - E)).reshape(nrow, CH)
    ew = jnp.pad(edge_weight, (0, E_PAD - E)).reshape(nrow, CH)
    et = jnp.pad(edge_type, (0, E_PAD - E),
                 constant_values=-1).reshape(nrow, CH)
    seed = jnp.asarray(seed_node_id, jnp.int32).reshape(1)
    bL0 = jnp.stack([b00, b10]); gL0 = jnp.stack([g00, g10])
    beL0 = jnp.stack([be00, be10])
    WL1 = jnp.stack([W01, W11])
    bL1 = jnp.stack([b01, b11]); gL1 = jnp.stack([g01, g11])
    beL1 = jnp.stack([be01, be11])
    Wp1ab = jnp.stack([Wp1[0:H], Wp1[H:2 * H]])
    Wp1cd = jnp.stack([Wp1[2 * H:3 * H], Wp1[3 * H:4 * H]])

    deg, ewm0, ewm1 = _sc_deg(dst, ew, et)
    deg4 = deg.reshape(NCORE, 2, NPAD)

    h = _tc_call(
        _tc_h_body, grid,
        [_row_spec(128), _full_spec((128, H)), _full_spec((H,))],
        _row_spec(H), jax.ShapeDtypeStruct((NPAD, H), f32))(xp, W_ft, b_ft)

    y, dis = _tc_call(
        _tc_b_body, grid,
        [_row_spec(H), pl.BlockSpec((NCORE, 2, ROWS), lambda i: (0, 0, i)),
         _full_spec((H, H)), _full_spec((H, H))],
        [_row_spec(H2), pl.BlockSpec((2, ROWS), lambda i: (0, i))],
        [jax.ShapeDtypeStruct((NPAD, H2), f32),
         jax.ShapeDtypeStruct((2, NPAD), f32)])(h, deg4, W00, W10)

    z = _sc_agg(src, dst, ewm0, ewm1, y)
    z = z.reshape(NCORE, NPAD, H2)

    o, y2 = _tc_call(
        _tc_c_body, grid,
        [_row2_spec(H2), _row_spec(H2),
         pl.BlockSpec((2, ROWS), lambda i: (0, i)),
         _full_spec((2, H)), _full_spec((2, H)),
         _full_spec((2, H)), _full_spec((2, H, H))],
        [_row2_spec(H), _row_spec(H2)],
        [jax.ShapeDtypeStruct((NCORE, NPAD, H), f32),
         jax.ShapeDtypeStruct((NPAD, H2), f32)])(
             z, y, dis, bL0, gL0, beL0, WL1)

    z2 = _sc_agg(src, dst, ewm0, ewm1, y2)
    z2 = z2.reshape(NCORE, NPAD, H2)

    x01 = _tc_call(
        _tc_d_body, grid,
        [_row2_spec(H2), _row_spec(H2), _row2_spec(H),
         pl.BlockSpec((2, ROWS), lambda i: (0, i)),
         _full_spec((2, H)), _full_spec((2, H)), _full_spec((2, H))],
        _row2_spec(H),
        jax.ShapeDtypeStruct((NCORE, NPAD, H), f32))(
            z2, y2, o, dis, bL1, gL1, beL1)

    sb = pl.pallas_call(
        _tc_s_body,
        grid=(1,),
        in_specs=[_full_spec((NCORE, NPAD, H)),
                  pl.BlockSpec(memory_space=pltpu.SMEM),
                  _full_spec((NCORE, H, H)), _full_spec((H,))],
        out_specs=_full_spec((1, H)),
        out_shape=jax.ShapeDtypeStruct((1, H), f32))(x01, seed, Wp1cd, bp1)

    res = _tc_call(
        _tc_e_body, grid,
        [_row2_spec(H), _full_spec((1, H)), _full_spec((NCORE, H, H)),
         _full_spec((H, H)), _full_spec((H,)), _full_spec((H, 1)),
         _full_spec((1,))],
        _row_spec(1),
        jax.ShapeDtypeStruct((NPAD, 1), f32))(
            x01, sb, Wp1ab, Wp2, bp2, Wp3, bp3)

    return res[:N, 0]
